# baseline (speedup 1.0000x reference)
import functools

import jax
import jax.numpy as jnp
from jax import lax
from jax.experimental import pallas as pl
from jax.experimental.pallas import tpu as pltpu
from jax.experimental.pallas import tpu_sc as plsc

_DIM = 64
_Q = 16
_EPS = 1e-8


def _scan_body(q_ref, x_ref, sim_ref, idx_ref, bsim_ref, bidx_ref):
    i = pl.program_id(0)
    nblk = pl.num_programs(0)
    blk = x_ref.shape[0]

    @pl.when(i == 0)
    def _init():
        bsim_ref[...] = jnp.full_like(bsim_ref, -jnp.inf)
        bidx_ref[...] = jnp.zeros_like(bidx_ref)

    q = q_ref[...]
    qn = q / (jnp.sqrt(jnp.sum(q * q, axis=1, keepdims=True)) + _EPS)

    x = x_ref[...]
    s = jax.lax.dot_general(qn, x, (((1,), (1,)), ((), ())),
                            preferred_element_type=jnp.float32)
    ones = jnp.ones((1, _DIM), jnp.float32)
    t = jax.lax.dot_general(ones, x * x, (((1,), (1,)), ((), ())),
                            preferred_element_type=jnp.float32)
    sims = s * (1.0 / (jnp.sqrt(t) + _EPS))

    bmax = jnp.max(sims, axis=1, keepdims=True)
    col = jax.lax.broadcasted_iota(jnp.int32, sims.shape, 1)
    lidx = jnp.min(jnp.where(sims >= bmax, col, blk), axis=1, keepdims=True)

    upd = bmax > bsim_ref[...]
    bsim_ref[...] = jnp.where(upd, bmax, bsim_ref[...])
    bidx_ref[...] = jnp.where(upd, i * blk + lidx, bidx_ref[...])

    @pl.when(i == nblk - 1)
    def _final():
        sim_ref[...] = bsim_ref[...]
        idx_ref[...] = bidx_ref[...]


def _scan(query, memories):
    cap = memories.shape[0]
    blk = 40000
    grid = cap // blk
    sim, idx = pl.pallas_call(
        _scan_body,
        grid=(grid,),
        in_specs=[
            pl.BlockSpec((_Q, _DIM), lambda i: (0, 0)),
            pl.BlockSpec((blk, _DIM), lambda i: (i, 0)),
        ],
        out_specs=[
            pl.BlockSpec((_Q, 1), lambda i: (0, 0)),
            pl.BlockSpec((_Q, 1), lambda i: (0, 0)),
        ],
        out_shape=[
            jax.ShapeDtypeStruct((_Q, 1), jnp.float32),
            jax.ShapeDtypeStruct((_Q, 1), jnp.int32),
        ],
        scratch_shapes=[
            pltpu.VMEM((_Q, 1), jnp.float32),
            pltpu.VMEM((_Q, 1), jnp.int32),
        ],
        compiler_params=pltpu.CompilerParams(
            dimension_semantics=("arbitrary",),
            vmem_limit_bytes=120 * 1024 * 1024,
        ),
    )(query, memories)
    return sim, idx


def _retrieve_body(mem_ref, idx_ref, sim_ref, wt_ref, b_ref, out_ref,
                   idx_v, row_v, sim_v, wt_v, b_v, ob0, ob1, sem):
    cid = lax.axis_index("c")
    sid = lax.axis_index("s")
    wid = sid * 2 + cid
    q = wid // 2
    h = wid % 2

    pltpu.sync_copy(idx_ref, idx_v)
    pltpu.sync_copy(sim_ref, sim_v)
    pltpu.sync_copy(wt_ref, wt_v)
    pltpu.sync_copy(b_ref, b_v)

    qvec = jnp.zeros((16,), jnp.int32) + q
    lanes = lax.broadcasted_iota(jnp.int32, (16,), 0)
    idx_val = jnp.max(jnp.where(lanes == qvec, idx_v[...],
                                jnp.int32(-2147483647)))
    base = (idx_val // 8) * 8
    sub = idx_val - base
    pltpu.async_copy(mem_ref.at[pl.ds(base, 8), :], row_v, sem).wait()
    svec = jnp.zeros((16,), jnp.int32) + sub

    sq = plsc.load_gather(sim_v, [qvec])
    gate = sq > 0.6
    zero = jnp.zeros((16,), jnp.float32)
    zvec = jnp.zeros((16,), jnp.int32)

    for hh in range(2):
        @pl.when(h == hh)
        def _half(hh=hh):
            c0 = 2 * hh
            c1 = 2 * hh + 1
            acc0 = jnp.zeros((16,), jnp.float32)
            acc1 = jnp.zeros((16,), jnp.float32)
            for d in range(_DIM):
                dvec = jnp.full((16,), d, jnp.int32)
                rs = plsc.load_gather(row_v, [svec, dvec])
                acc0 = acc0 + rs * wt_v[d, 16 * c0:16 * c0 + 16]
                acc1 = acc1 + rs * wt_v[d, 16 * c1:16 * c1 + 16]
            r0 = jnp.where(gate, acc0 + b_v[16 * c0:16 * c0 + 16], zero)
            r1 = jnp.where(gate, acc1 + b_v[16 * c1:16 * c1 + 16], zero)
            ob0[...] = r0
            ob1[...] = r1
            pltpu.sync_copy(ob0, out_ref.at[q, pl.ds(16 * c0, 16)])
            pltpu.sync_copy(ob1, out_ref.at[q, pl.ds(16 * c1, 16)])


def _retrieve(memories, idx1d, sim1d, wt, b_dec):
    mesh = plsc.VectorSubcoreMesh(core_axis_name="c", subcore_axis_name="s")
    k = functools.partial(
        pl.kernel,
        mesh=mesh,
        out_type=jax.ShapeDtypeStruct((_Q, _DIM), jnp.float32),
        scratch_types=[
            pltpu.VMEM((_Q,), jnp.int32),
            pltpu.VMEM((8, _DIM), jnp.float32),
            pltpu.VMEM((_Q,), jnp.float32),
            pltpu.VMEM((_DIM, _DIM), jnp.float32),
            pltpu.VMEM((_DIM,), jnp.float32),
            pltpu.VMEM((16,), jnp.float32),
            pltpu.VMEM((16,), jnp.float32),
            pltpu.SemaphoreType.DMA,
        ],
        compiler_params=pltpu.CompilerParams(needs_layout_passes=False),
    )(_retrieve_body)
    return k(memories, idx1d, sim1d, wt, b_dec)


def kernel(query, memories, W_dec, b_dec):
    sim, idx = _scan(query, memories)
    out = _retrieve(memories, idx.reshape(_Q), sim.reshape(_Q),
                    W_dec.T, b_dec)
    return out


# speedup vs baseline: 1.0048x; 1.0048x over previous
import functools

import jax
import jax.numpy as jnp
from jax import lax
from jax.experimental import pallas as pl
from jax.experimental.pallas import tpu as pltpu
from jax.experimental.pallas import tpu_sc as plsc

_DIM = 64
_Q = 16
_EPS = 1e-8


def _scan_body(q_ref, x_ref, sim_ref, idx_ref, bsim_ref, bidx_ref):
    i = pl.program_id(0)
    nblk = pl.num_programs(0)
    blk = x_ref.shape[0]

    @pl.when(i == 0)
    def _init():
        bsim_ref[...] = jnp.full_like(bsim_ref, -jnp.inf)
        bidx_ref[...] = jnp.zeros_like(bidx_ref)

    q = q_ref[...]
    qn = q / (jnp.sqrt(jnp.sum(q * q, axis=1, keepdims=True)) + _EPS)

    x = x_ref[...]
    s = jax.lax.dot_general(qn, x, (((1,), (1,)), ((), ())),
                            preferred_element_type=jnp.float32)
    ones = jnp.ones((1, _DIM), jnp.float32)
    t = jax.lax.dot_general(ones, x * x, (((1,), (1,)), ((), ())),
                            preferred_element_type=jnp.float32)
    sims = s * (1.0 / (jnp.sqrt(t) + _EPS))

    bmax = jnp.max(sims, axis=1, keepdims=True)
    col = jax.lax.broadcasted_iota(jnp.int32, sims.shape, 1)
    lidx = jnp.min(jnp.where(sims >= bmax, col, blk), axis=1, keepdims=True)

    upd = bmax > bsim_ref[...]
    bsim_ref[...] = jnp.where(upd, bmax, bsim_ref[...])
    bidx_ref[...] = jnp.where(upd, i * blk + lidx, bidx_ref[...])

    @pl.when(i == nblk - 1)
    def _final():
        sim_ref[...] = bsim_ref[...]
        idx_ref[...] = bidx_ref[...]


def _scan(query, memories):
    cap = memories.shape[0]
    blk = 40000
    grid = cap // blk
    sim, idx = pl.pallas_call(
        _scan_body,
        grid=(grid,),
        in_specs=[
            pl.BlockSpec((_Q, _DIM), lambda i: (0, 0)),
            pl.BlockSpec((blk, _DIM), lambda i: (i, 0)),
        ],
        out_specs=[
            pl.BlockSpec((_Q, 1), lambda i: (0, 0)),
            pl.BlockSpec((_Q, 1), lambda i: (0, 0)),
        ],
        out_shape=[
            jax.ShapeDtypeStruct((_Q, 1), jnp.float32),
            jax.ShapeDtypeStruct((_Q, 1), jnp.int32),
        ],
        scratch_shapes=[
            pltpu.VMEM((_Q, 1), jnp.float32),
            pltpu.VMEM((_Q, 1), jnp.int32),
        ],
        compiler_params=pltpu.CompilerParams(
            dimension_semantics=("arbitrary",),
            vmem_limit_bytes=120 * 1024 * 1024,
        ),
    )(query, memories)
    return sim, idx


def _retrieve_body(mem_ref, idx_ref, sim_ref, wt_ref, b_ref, out_ref,
                   idx_v, row_v, sim_v, wt_v, b_v, ob0, ob1, sem):
    cid = lax.axis_index("c")
    sid = lax.axis_index("s")
    wid = sid * 2 + cid
    q = wid // 2
    h = wid % 2

    ci = pltpu.async_copy(idx_ref, idx_v, sem)
    cs = pltpu.async_copy(sim_ref, sim_v, sem)
    cw = pltpu.async_copy(wt_ref, wt_v, sem)
    cb = pltpu.async_copy(b_ref, b_v, sem)
    ci.wait()

    qvec = jnp.zeros((16,), jnp.int32) + q
    lanes = lax.broadcasted_iota(jnp.int32, (16,), 0)
    idx_val = jnp.max(jnp.where(lanes == qvec, idx_v[...],
                                jnp.int32(-2147483647)))
    base = (idx_val // 8) * 8
    sub = idx_val - base
    cr = pltpu.async_copy(mem_ref.at[pl.ds(base, 8), :], row_v, sem)
    cs.wait()
    cw.wait()
    cb.wait()
    cr.wait()
    svec = jnp.zeros((16,), jnp.int32) + sub

    sq = plsc.load_gather(sim_v, [qvec])
    gate = sq > 0.6
    zero = jnp.zeros((16,), jnp.float32)
    zvec = jnp.zeros((16,), jnp.int32)

    for hh in range(2):
        @pl.when(h == hh)
        def _half(hh=hh):
            c0 = 2 * hh
            c1 = 2 * hh + 1
            acc0 = jnp.zeros((16,), jnp.float32)
            acc1 = jnp.zeros((16,), jnp.float32)
            for d in range(_DIM):
                dvec = jnp.full((16,), d, jnp.int32)
                rs = plsc.load_gather(row_v, [svec, dvec])
                acc0 = acc0 + rs * wt_v[d, 16 * c0:16 * c0 + 16]
                acc1 = acc1 + rs * wt_v[d, 16 * c1:16 * c1 + 16]
            r0 = jnp.where(gate, acc0 + b_v[16 * c0:16 * c0 + 16], zero)
            r1 = jnp.where(gate, acc1 + b_v[16 * c1:16 * c1 + 16], zero)
            ob0[...] = r0
            ob1[...] = r1
            pltpu.sync_copy(ob0, out_ref.at[q, pl.ds(16 * c0, 16)])
            pltpu.sync_copy(ob1, out_ref.at[q, pl.ds(16 * c1, 16)])


def _retrieve(memories, idx1d, sim1d, wt, b_dec):
    mesh = plsc.VectorSubcoreMesh(core_axis_name="c", subcore_axis_name="s")
    k = functools.partial(
        pl.kernel,
        mesh=mesh,
        out_type=jax.ShapeDtypeStruct((_Q, _DIM), jnp.float32),
        scratch_types=[
            pltpu.VMEM((_Q,), jnp.int32),
            pltpu.VMEM((8, _DIM), jnp.float32),
            pltpu.VMEM((_Q,), jnp.float32),
            pltpu.VMEM((_DIM, _DIM), jnp.float32),
            pltpu.VMEM((_DIM,), jnp.float32),
            pltpu.VMEM((16,), jnp.float32),
            pltpu.VMEM((16,), jnp.float32),
            pltpu.SemaphoreType.DMA,
        ],
        compiler_params=pltpu.CompilerParams(needs_layout_passes=False),
    )(_retrieve_body)
    return k(memories, idx1d, sim1d, wt, b_dec)


def kernel(query, memories, W_dec, b_dec):
    sim, idx = _scan(query, memories)
    out = _retrieve(memories, idx.reshape(_Q), sim.reshape(_Q),
                    W_dec.T, b_dec)
    return out
